# baseline (speedup 1.0000x reference)
import functools

import jax
import jax.numpy as jnp
from jax import lax
from jax.experimental import pallas as pl
from jax.experimental.pallas import tpu as pltpu
from jax.experimental.pallas import tpu_sc as plsc

N = 10000
E = 320000
D = 128
NT = 64

NC = 2
NS = 16
NW = NC * NS

HALF = 4992
H_ROWS = 5016
H_TRASH = 5008

KB = 32
PER_W = 20480
PAD_SC = NS * PER_W
STAGE_E = 640
STAGE_CH = STAGE_E // KB
NSTAGES = PER_W // STAGE_E

ROWS_PER_TILE = 632
ACC_ROWS = NS * ROWS_PER_TILE
TRASH = N

_R = 1000


def _make_sc_segsum():
    mesh = plsc.VectorSubcoreMesh(core_axis_name="c", subcore_axis_name="s")

    @functools.partial(
        pl.kernel,
        out_type=jax.ShapeDtypeStruct((NC * ACC_ROWS, D), jnp.float32),
        mesh=mesh,
        scratch_types=[
            pltpu.VMEM((STAGE_E,), jnp.int32),
            pltpu.VMEM((STAGE_E,), jnp.int32),
            pltpu.VMEM((2, KB, D), jnp.float32),
            pltpu.VMEM_SHARED((H_ROWS, D), jnp.float32),
            pltpu.VMEM_SHARED((ACC_ROWS, D), jnp.float32),
            pltpu.SemaphoreType.DMA,
            pltpu.SemaphoreType.DMA,
        ],
    )
    def segsum(h_hbm, src_hbm, dst_hbm, zeros_hbm, out_hbm,
               src_v, dst_v, rows, hsp, acc, sem_a, sem_b):
        c = lax.axis_index("c")
        s = lax.axis_index("s")
        sems = (sem_a, sem_b)

        pltpu.sync_copy(zeros_hbm, acc.at[pl.ds(s * ROWS_PER_TILE, ROWS_PER_TILE)])
        base_h = c * HALF
        pltpu.sync_copy(h_hbm.at[pl.ds(base_h + s * 312, 312)],
                        hsp.at[pl.ds(s * 312, 312)])

        @pl.when(s == 0)
        def _tail():
            pltpu.sync_copy(h_hbm.at[pl.ds(base_h + 4992, 16)],
                            hsp.at[pl.ds(4992, 16)])

        plsc.subcore_barrier()

        ebase = c * PAD_SC + s * PER_W

        def stage_body(st, carry):
            off = ebase + st * STAGE_E
            pltpu.sync_copy(src_hbm.at[pl.ds(off, STAGE_E)], src_v)
            pltpu.sync_copy(dst_hbm.at[pl.ds(off, STAGE_E)], dst_v)
            pltpu.async_copy(hsp.at[src_v.at[pl.ds(0, KB)]], rows.at[0], sem_a)

            def body(g, carry2):
                for b in range(2):
                    j = 2 * g + b
                    pltpu.make_async_copy(
                        hsp.at[src_v.at[pl.ds(j * KB, KB)]],
                        rows.at[b], sems[b]).wait()
                    pltpu.sync_copy(
                        rows.at[b], acc.at[dst_v.at[pl.ds(j * KB, KB)]],
                        add=True)
                    nxt = j + 1
                    bn = (b + 1) % 2

                    @pl.when(nxt < STAGE_CH)
                    def _():
                        pltpu.async_copy(
                            hsp.at[src_v.at[pl.ds(nxt * KB, KB)]],
                            rows.at[bn], sems[bn])
                return carry2

            lax.fori_loop(0, STAGE_CH // 2, body, 0)
            return carry

        lax.fori_loop(0, NSTAGES, stage_body, 0)
        plsc.subcore_barrier()

        pltpu.sync_copy(
            acc.at[pl.ds(s * ROWS_PER_TILE, ROWS_PER_TILE)],
            out_hbm.at[pl.ds((c * ACC_ROWS + s * ROWS_PER_TILE), ROWS_PER_TILE)])

    return segsum


_sc_segsum = _make_sc_segsum()


def _tc_layer_body(p0, p1, h, wr, ws, b, o):
    agg = p0[...] + p1[...]
    acc = jnp.dot(agg, wr[...], preferred_element_type=jnp.float32)
    acc += jnp.dot(h[...], ws[...], preferred_element_type=jnp.float32)
    o[...] = jnp.maximum(acc + b[...], 0.0)


_tc_layer = pl.pallas_call(
    _tc_layer_body,
    grid=(N // _R,),
    in_specs=[
        pl.BlockSpec((_R, D), lambda i: (i, 0)),
        pl.BlockSpec((_R, D), lambda i: (i, 0)),
        pl.BlockSpec((_R, D), lambda i: (i, 0)),
        pl.BlockSpec((D, D), lambda i: (0, 0)),
        pl.BlockSpec((D, D), lambda i: (0, 0)),
        pl.BlockSpec((1, D), lambda i: (0, 0)),
    ],
    out_specs=pl.BlockSpec((_R, D), lambda i: (i, 0)),
    out_shape=jax.ShapeDtypeStruct((N, D), jnp.float32),
)


def _tc_final_body(p0, p1, h, wr, ws, b, wc1, bc1, wc2, bc2, o):
    agg = p0[...] + p1[...]
    acc = jnp.dot(agg, wr[...], preferred_element_type=jnp.float32)
    acc += jnp.dot(h[...], ws[...], preferred_element_type=jnp.float32)
    h3 = jnp.maximum(acc + b[...], 0.0)
    hc = jnp.maximum(
        jnp.dot(h3, wc1[...], preferred_element_type=jnp.float32) + bc1[...], 0.0)
    o[...] = jnp.dot(hc, wc2[...], preferred_element_type=jnp.float32) + bc2[...]


_tc_final = pl.pallas_call(
    _tc_final_body,
    grid=(N // _R,),
    in_specs=[
        pl.BlockSpec((_R, D), lambda i: (i, 0)),
        pl.BlockSpec((_R, D), lambda i: (i, 0)),
        pl.BlockSpec((_R, D), lambda i: (i, 0)),
        pl.BlockSpec((D, D), lambda i: (0, 0)),
        pl.BlockSpec((D, D), lambda i: (0, 0)),
        pl.BlockSpec((1, D), lambda i: (0, 0)),
        pl.BlockSpec((D, D), lambda i: (0, 0)),
        pl.BlockSpec((1, D), lambda i: (0, 0)),
        pl.BlockSpec((D, NT), lambda i: (0, 0)),
        pl.BlockSpec((1, NT), lambda i: (0, 0)),
    ],
    out_specs=pl.BlockSpec((_R, NT), lambda i: (i, 0)),
    out_shape=jax.ShapeDtypeStruct((N, NT), jnp.float32),
)


def _prep_indices(edge_index):
    src = edge_index[0].astype(jnp.int32)
    dst = edge_index[1].astype(jnp.int32)

    def mask_for(cid):
        if cid == 0:
            hit = src < HALF
            lsrc = src
        else:
            hit = src >= HALF
            lsrc = src - HALF
        ms = jnp.where(hit, lsrc, H_TRASH)
        md = jnp.where(hit, dst, TRASH)
        ms = jnp.concatenate(
            [ms.reshape(NS, E // NS),
             jnp.full((NS, PER_W - E // NS), H_TRASH, jnp.int32)], axis=1)
        md = jnp.concatenate(
            [md.reshape(NS, E // NS),
             jnp.full((NS, PER_W - E // NS), TRASH, jnp.int32)], axis=1)
        return ms.reshape(-1), md.reshape(-1)

    s0, d0 = mask_for(0)
    s1, d1 = mask_for(1)
    return jnp.concatenate([s0, s1]), jnp.concatenate([d0, d1])


def kernel(x, edge_index, Wr0, Ws0, b0, Wr1, Ws1, b1, Wr2, Ws2, b2,
           Wc1, bc1, Wc2, bc2):
    src_p, dst_p = _prep_indices(edge_index)
    zeros = jnp.zeros((ROWS_PER_TILE, D), jnp.float32)

    b0r = b0.reshape(1, D)
    b1r = b1.reshape(1, D)
    b2r = b2.reshape(1, D)
    bc1r = bc1.reshape(1, D)
    bc2r = bc2.reshape(1, NT)

    h = x
    for (wr, ws, br) in ((Wr0, Ws0, b0r), (Wr1, Ws1, b1r)):
        parts = _sc_segsum(h, src_p, dst_p, zeros)
        p0 = parts[:N]
        p1 = parts[ACC_ROWS:ACC_ROWS + N]
        h = _tc_layer(p0, p1, h, wr, ws, br)

    parts = _sc_segsum(h, src_p, dst_p, zeros)
    p0 = parts[:N]
    p1 = parts[ACC_ROWS:ACC_ROWS + N]
    return _tc_final(p0, p1, h, Wr2, Ws2, b2r, Wc1, bc1r, Wc2, bc2r)


# speedup vs baseline: 1.0578x; 1.0578x over previous
import functools

import jax
import jax.numpy as jnp
from jax import lax
from jax.experimental import pallas as pl
from jax.experimental.pallas import tpu as pltpu
from jax.experimental.pallas import tpu_sc as plsc

N = 10000
E = 320000
D = 128
NT = 64

NC = 2
NS = 16
NW = NC * NS

HALF = 4992
H_ROWS = 5016
H_TRASH = 5008

KB = 32
PER_W = 20480
PAD_SC = NS * PER_W
STAGE_E = 320
STAGE_CH = STAGE_E // KB
NSTAGES = PER_W // STAGE_E

ROWS_PER_TILE = 632
ACC_ROWS = NS * ROWS_PER_TILE
TRASH = N

_R = 1000


def _make_sc_segsum():
    mesh = plsc.VectorSubcoreMesh(core_axis_name="c", subcore_axis_name="s")

    @functools.partial(
        pl.kernel,
        out_type=jax.ShapeDtypeStruct((NC * ACC_ROWS, D), jnp.float32),
        mesh=mesh,
        scratch_types=[
            pltpu.VMEM((STAGE_E,), jnp.int32),
            pltpu.VMEM((STAGE_E,), jnp.int32),
            pltpu.VMEM((STAGE_E,), jnp.int32),
            pltpu.VMEM((STAGE_E,), jnp.int32),
            pltpu.VMEM((2, KB, D), jnp.float32),
            pltpu.VMEM_SHARED((H_ROWS, D), jnp.float32),
            pltpu.VMEM_SHARED((ACC_ROWS, D), jnp.float32),
            pltpu.SemaphoreType.DMA,
            pltpu.SemaphoreType.DMA,
            pltpu.SemaphoreType.DMA,
            pltpu.SemaphoreType.DMA,
        ],
    )
    def segsum(h_hbm, src_hbm, dst_hbm, zeros_hbm, out_hbm,
               src_a, dst_a, src_b, dst_b, rows, hsp, acc,
               sem_a, sem_b, sem_ia, sem_ib):
        c = lax.axis_index("c")
        s = lax.axis_index("s")
        sems = (sem_a, sem_b)

        pltpu.sync_copy(zeros_hbm, acc.at[pl.ds(s * ROWS_PER_TILE, ROWS_PER_TILE)])
        base_h = c * HALF
        pltpu.sync_copy(h_hbm.at[pl.ds(base_h + s * 312, 312)],
                        hsp.at[pl.ds(s * 312, 312)])

        @pl.when(s == 0)
        def _tail():
            pltpu.sync_copy(h_hbm.at[pl.ds(base_h + 4992, 16)],
                            hsp.at[pl.ds(4992, 16)])

        plsc.subcore_barrier()

        ebase = c * PAD_SC + s * PER_W
        idx_sets = ((src_a, dst_a, sem_ia), (src_b, dst_b, sem_ib))

        def load_idx(st, sset):
            sv, dv, sem = sset
            off = ebase + st * STAGE_E
            pltpu.async_copy(src_hbm.at[pl.ds(off, STAGE_E)], sv, sem)
            pltpu.async_copy(dst_hbm.at[pl.ds(off, STAGE_E)], dv, sem)

        def wait_idx(st, sset):
            sv, dv, sem = sset
            off = ebase + st * STAGE_E
            pltpu.make_async_copy(src_hbm.at[pl.ds(off, STAGE_E)], sv, sem).wait()
            pltpu.make_async_copy(dst_hbm.at[pl.ds(off, STAGE_E)], dv, sem).wait()

        def run_stage(sset):
            sv, dv, _ = sset
            pltpu.async_copy(hsp.at[sv.at[pl.ds(0, KB)]], rows.at[0], sem_a)

            def body(g, carry2):
                for b in range(2):
                    j = 2 * g + b
                    pltpu.make_async_copy(
                        hsp.at[sv.at[pl.ds(j * KB, KB)]],
                        rows.at[b], sems[b]).wait()
                    pltpu.sync_copy(
                        rows.at[b], acc.at[dv.at[pl.ds(j * KB, KB)]],
                        add=True)
                    nxt = j + 1
                    bn = (b + 1) % 2

                    @pl.when(nxt < STAGE_CH)
                    def _():
                        pltpu.async_copy(
                            hsp.at[sv.at[pl.ds(nxt * KB, KB)]],
                            rows.at[bn], sems[bn])
                return carry2

            lax.fori_loop(0, STAGE_CH // 2, body, 0)

        load_idx(0, idx_sets[0])

        def pair_body(t, carry):
            st0 = 2 * t
            wait_idx(st0, idx_sets[0])
            load_idx(st0 + 1, idx_sets[1])
            run_stage(idx_sets[0])
            wait_idx(st0 + 1, idx_sets[1])

            @pl.when(st0 + 2 < NSTAGES)
            def _():
                load_idx(st0 + 2, idx_sets[0])

            run_stage(idx_sets[1])
            return carry

        lax.fori_loop(0, NSTAGES // 2, pair_body, 0)
        plsc.subcore_barrier()

        pltpu.sync_copy(
            acc.at[pl.ds(s * ROWS_PER_TILE, ROWS_PER_TILE)],
            out_hbm.at[pl.ds((c * ACC_ROWS + s * ROWS_PER_TILE), ROWS_PER_TILE)])

    return segsum


_sc_segsum = _make_sc_segsum()


def _tc_layer_body(p0, p1, h, wr, ws, b, o):
    agg = p0[...] + p1[...]
    acc = jnp.dot(agg, wr[...], preferred_element_type=jnp.float32)
    acc += jnp.dot(h[...], ws[...], preferred_element_type=jnp.float32)
    o[...] = jnp.maximum(acc + b[...], 0.0)


_tc_layer = pl.pallas_call(
    _tc_layer_body,
    grid=(N // _R,),
    in_specs=[
        pl.BlockSpec((_R, D), lambda i: (i, 0)),
        pl.BlockSpec((_R, D), lambda i: (i, 0)),
        pl.BlockSpec((_R, D), lambda i: (i, 0)),
        pl.BlockSpec((D, D), lambda i: (0, 0)),
        pl.BlockSpec((D, D), lambda i: (0, 0)),
        pl.BlockSpec((1, D), lambda i: (0, 0)),
    ],
    out_specs=pl.BlockSpec((_R, D), lambda i: (i, 0)),
    out_shape=jax.ShapeDtypeStruct((N, D), jnp.float32),
)


def _tc_final_body(p0, p1, h, wr, ws, b, wc1, bc1, wc2, bc2, o):
    agg = p0[...] + p1[...]
    acc = jnp.dot(agg, wr[...], preferred_element_type=jnp.float32)
    acc += jnp.dot(h[...], ws[...], preferred_element_type=jnp.float32)
    h3 = jnp.maximum(acc + b[...], 0.0)
    hc = jnp.maximum(
        jnp.dot(h3, wc1[...], preferred_element_type=jnp.float32) + bc1[...], 0.0)
    o[...] = jnp.dot(hc, wc2[...], preferred_element_type=jnp.float32) + bc2[...]


_tc_final = pl.pallas_call(
    _tc_final_body,
    grid=(N // _R,),
    in_specs=[
        pl.BlockSpec((_R, D), lambda i: (i, 0)),
        pl.BlockSpec((_R, D), lambda i: (i, 0)),
        pl.BlockSpec((_R, D), lambda i: (i, 0)),
        pl.BlockSpec((D, D), lambda i: (0, 0)),
        pl.BlockSpec((D, D), lambda i: (0, 0)),
        pl.BlockSpec((1, D), lambda i: (0, 0)),
        pl.BlockSpec((D, D), lambda i: (0, 0)),
        pl.BlockSpec((1, D), lambda i: (0, 0)),
        pl.BlockSpec((D, NT), lambda i: (0, 0)),
        pl.BlockSpec((1, NT), lambda i: (0, 0)),
    ],
    out_specs=pl.BlockSpec((_R, NT), lambda i: (i, 0)),
    out_shape=jax.ShapeDtypeStruct((N, NT), jnp.float32),
)


def _prep_indices(edge_index):
    src = edge_index[0].astype(jnp.int32)
    dst = edge_index[1].astype(jnp.int32)

    def mask_for(cid):
        if cid == 0:
            hit = src < HALF
            lsrc = src
        else:
            hit = src >= HALF
            lsrc = src - HALF
        ms = jnp.where(hit, lsrc, H_TRASH)
        md = jnp.where(hit, dst, TRASH)
        ms = jnp.concatenate(
            [ms.reshape(NS, E // NS),
             jnp.full((NS, PER_W - E // NS), H_TRASH, jnp.int32)], axis=1)
        md = jnp.concatenate(
            [md.reshape(NS, E // NS),
             jnp.full((NS, PER_W - E // NS), TRASH, jnp.int32)], axis=1)
        return ms.reshape(-1), md.reshape(-1)

    s0, d0 = mask_for(0)
    s1, d1 = mask_for(1)
    return jnp.concatenate([s0, s1]), jnp.concatenate([d0, d1])


def kernel(x, edge_index, Wr0, Ws0, b0, Wr1, Ws1, b1, Wr2, Ws2, b2,
           Wc1, bc1, Wc2, bc2):
    src_p, dst_p = _prep_indices(edge_index)
    zeros = jnp.zeros((ROWS_PER_TILE, D), jnp.float32)

    b0r = b0.reshape(1, D)
    b1r = b1.reshape(1, D)
    b2r = b2.reshape(1, D)
    bc1r = bc1.reshape(1, D)
    bc2r = bc2.reshape(1, NT)

    h = x
    for (wr, ws, br) in ((Wr0, Ws0, b0r), (Wr1, Ws1, b1r)):
        parts = _sc_segsum(h, src_p, dst_p, zeros)
        p0 = parts[:N]
        p1 = parts[ACC_ROWS:ACC_ROWS + N]
        h = _tc_layer(p0, p1, h, wr, ws, br)

    parts = _sc_segsum(h, src_p, dst_p, zeros)
    p0 = parts[:N]
    p1 = parts[ACC_ROWS:ACC_ROWS + N]
    return _tc_final(p0, p1, h, Wr2, Ws2, b2r, Wc1, bc1r, Wc2, bc2r)


# speedup vs baseline: 1.0891x; 1.0296x over previous
import functools

import jax
import jax.numpy as jnp
from jax import lax
from jax.experimental import pallas as pl
from jax.experimental.pallas import tpu as pltpu
from jax.experimental.pallas import tpu_sc as plsc

N = 10000
E = 320000
D = 128
NT = 64

NC = 2
NS = 16
NW = NC * NS

HALF = 4992
H_ROWS = 5016
H_TRASH = 5008

KB = 32
PER_W = 20480
PAD_SC = NS * PER_W
STAGE_E = 320
STAGE_CH = STAGE_E // KB
NSTAGES = PER_W // STAGE_E

ROWS_PER_TILE = 632
ACC_ROWS = NS * ROWS_PER_TILE
TRASH = N

_R = 1000


def _make_sc_segsum():
    mesh = plsc.VectorSubcoreMesh(core_axis_name="c", subcore_axis_name="s")

    @functools.partial(
        pl.kernel,
        out_type=jax.ShapeDtypeStruct((NC * ACC_ROWS, D), jnp.float32),
        mesh=mesh,
        scratch_types=[
            pltpu.VMEM((STAGE_E,), jnp.int32),
            pltpu.VMEM((STAGE_E,), jnp.int32),
            pltpu.VMEM((STAGE_E,), jnp.int32),
            pltpu.VMEM((STAGE_E,), jnp.int32),
            pltpu.VMEM((2, KB, D), jnp.float32),
            pltpu.VMEM_SHARED((H_ROWS, D), jnp.float32),
            pltpu.VMEM_SHARED((ACC_ROWS, D), jnp.float32),
            pltpu.SemaphoreType.DMA,
            pltpu.SemaphoreType.DMA,
            pltpu.SemaphoreType.DMA,
            pltpu.SemaphoreType.DMA,
        ],
    )
    def segsum(h_hbm, src_hbm, dst_hbm, zeros_hbm, out_hbm,
               src_a, dst_a, src_b, dst_b, rows, hsp, acc,
               sem_a, sem_b, sem_ia, sem_ib):
        c = lax.axis_index("c")
        s = lax.axis_index("s")
        sems = (sem_a, sem_b)

        pltpu.sync_copy(zeros_hbm, acc.at[pl.ds(s * ROWS_PER_TILE, ROWS_PER_TILE)])
        base_h = c * HALF
        pltpu.sync_copy(h_hbm.at[pl.ds(base_h + s * 312, 312)],
                        hsp.at[pl.ds(s * 312, 312)])

        @pl.when(s == 0)
        def _tail():
            pltpu.sync_copy(h_hbm.at[pl.ds(base_h + 4992, 16)],
                            hsp.at[pl.ds(4992, 16)])

        plsc.subcore_barrier()

        ebase = c * PAD_SC + s * PER_W
        idx_sets = ((src_a, dst_a, sem_ia), (src_b, dst_b, sem_ib))

        def load_idx(st, sset):
            sv, dv, sem = sset
            off = ebase + st * STAGE_E
            pltpu.async_copy(src_hbm.at[pl.ds(off, STAGE_E)], sv, sem)
            pltpu.async_copy(dst_hbm.at[pl.ds(off, STAGE_E)], dv, sem)

        def wait_idx(st, sset):
            sv, dv, sem = sset
            off = ebase + st * STAGE_E
            pltpu.make_async_copy(src_hbm.at[pl.ds(off, STAGE_E)], sv, sem).wait()
            pltpu.make_async_copy(dst_hbm.at[pl.ds(off, STAGE_E)], dv, sem).wait()

        def run_stage(sset):
            sv, dv, _ = sset
            pltpu.async_copy(hsp.at[sv.at[pl.ds(0, KB)]], rows.at[0], sem_a)

            def body(g, carry2):
                for b in range(2):
                    j = 2 * g + b
                    pltpu.make_async_copy(
                        hsp.at[sv.at[pl.ds(j * KB, KB)]],
                        rows.at[b], sems[b]).wait()
                    pltpu.sync_copy(
                        rows.at[b], acc.at[dv.at[pl.ds(j * KB, KB)]],
                        add=True)
                    nxt = j + 1
                    bn = (b + 1) % 2

                    @pl.when(nxt < STAGE_CH)
                    def _():
                        pltpu.async_copy(
                            hsp.at[sv.at[pl.ds(nxt * KB, KB)]],
                            rows.at[bn], sems[bn])
                return carry2

            lax.fori_loop(0, STAGE_CH // 2, body, 0)

        load_idx(0, idx_sets[0])

        def pair_body(t, carry):
            st0 = 2 * t
            wait_idx(st0, idx_sets[0])
            load_idx(st0 + 1, idx_sets[1])
            run_stage(idx_sets[0])
            wait_idx(st0 + 1, idx_sets[1])

            @pl.when(st0 + 2 < NSTAGES)
            def _():
                load_idx(st0 + 2, idx_sets[0])

            run_stage(idx_sets[1])
            return carry

        lax.fori_loop(0, NSTAGES // 2, pair_body, 0)
        plsc.subcore_barrier()

        pltpu.sync_copy(
            acc.at[pl.ds(s * ROWS_PER_TILE, ROWS_PER_TILE)],
            out_hbm.at[pl.ds((c * ACC_ROWS + s * ROWS_PER_TILE), ROWS_PER_TILE)])

    return segsum


_sc_segsum = _make_sc_segsum()


def _tc_layer_body(p0, p1, h, wr, ws, b, o):
    agg = p0[...] + p1[...]
    acc = jnp.dot(agg, wr[...], preferred_element_type=jnp.float32)
    acc += jnp.dot(h[...], ws[...], preferred_element_type=jnp.float32)
    o[...] = jnp.maximum(acc + b[...], 0.0)


_tc_layer = pl.pallas_call(
    _tc_layer_body,
    grid=(N // _R,),
    in_specs=[
        pl.BlockSpec((_R, D), lambda i: (i, 0)),
        pl.BlockSpec((_R, D), lambda i: (i, 0)),
        pl.BlockSpec((_R, D), lambda i: (i, 0)),
        pl.BlockSpec((D, D), lambda i: (0, 0)),
        pl.BlockSpec((D, D), lambda i: (0, 0)),
        pl.BlockSpec((1, D), lambda i: (0, 0)),
    ],
    out_specs=pl.BlockSpec((_R, D), lambda i: (i, 0)),
    out_shape=jax.ShapeDtypeStruct((N, D), jnp.float32),
)


def _tc_final_body(p0, p1, h, wr, ws, b, wc1, bc1, wc2, bc2, o):
    agg = p0[...] + p1[...]
    acc = jnp.dot(agg, wr[...], preferred_element_type=jnp.float32)
    acc += jnp.dot(h[...], ws[...], preferred_element_type=jnp.float32)
    h3 = jnp.maximum(acc + b[...], 0.0)
    hc = jnp.maximum(
        jnp.dot(h3, wc1[...], preferred_element_type=jnp.float32) + bc1[...], 0.0)
    o[...] = jnp.dot(hc, wc2[...], preferred_element_type=jnp.float32) + bc2[...]


_tc_final = pl.pallas_call(
    _tc_final_body,
    grid=(N // _R,),
    in_specs=[
        pl.BlockSpec((_R, D), lambda i: (i, 0)),
        pl.BlockSpec((_R, D), lambda i: (i, 0)),
        pl.BlockSpec((_R, D), lambda i: (i, 0)),
        pl.BlockSpec((D, D), lambda i: (0, 0)),
        pl.BlockSpec((D, D), lambda i: (0, 0)),
        pl.BlockSpec((1, D), lambda i: (0, 0)),
        pl.BlockSpec((D, D), lambda i: (0, 0)),
        pl.BlockSpec((1, D), lambda i: (0, 0)),
        pl.BlockSpec((D, NT), lambda i: (0, 0)),
        pl.BlockSpec((1, NT), lambda i: (0, 0)),
    ],
    out_specs=pl.BlockSpec((_R, NT), lambda i: (i, 0)),
    out_shape=jax.ShapeDtypeStruct((N, NT), jnp.float32),
)


def _prep_indices(edge_index):
    src = edge_index[0].astype(jnp.int32)
    dst = edge_index[1].astype(jnp.int32)

    def mask_for(cid):
        if cid == 0:
            hit = src < HALF
            lsrc = src
        else:
            hit = src >= HALF
            lsrc = src - HALF
        eid = jnp.arange(E, dtype=jnp.int32)
        ms = jnp.where(hit, lsrc, H_TRASH + (eid % (H_ROWS - H_TRASH)))
        md = jnp.where(hit, dst, TRASH + (eid % (ACC_ROWS - TRASH)))
        ms = jnp.concatenate(
            [ms.reshape(NS, E // NS),
             jnp.full((NS, PER_W - E // NS), H_TRASH, jnp.int32)], axis=1)
        md = jnp.concatenate(
            [md.reshape(NS, E // NS),
             jnp.full((NS, PER_W - E // NS), TRASH, jnp.int32)], axis=1)
        return ms.reshape(-1), md.reshape(-1)

    s0, d0 = mask_for(0)
    s1, d1 = mask_for(1)
    return jnp.concatenate([s0, s1]), jnp.concatenate([d0, d1])


def kernel(x, edge_index, Wr0, Ws0, b0, Wr1, Ws1, b1, Wr2, Ws2, b2,
           Wc1, bc1, Wc2, bc2):
    src_p, dst_p = _prep_indices(edge_index)
    zeros = jnp.zeros((ROWS_PER_TILE, D), jnp.float32)

    b0r = b0.reshape(1, D)
    b1r = b1.reshape(1, D)
    b2r = b2.reshape(1, D)
    bc1r = bc1.reshape(1, D)
    bc2r = bc2.reshape(1, NT)

    h = x
    for (wr, ws, br) in ((Wr0, Ws0, b0r), (Wr1, Ws1, b1r)):
        parts = _sc_segsum(h, src_p, dst_p, zeros)
        p0 = parts[:N]
        p1 = parts[ACC_ROWS:ACC_ROWS + N]
        h = _tc_layer(p0, p1, h, wr, ws, br)

    parts = _sc_segsum(h, src_p, dst_p, zeros)
    p0 = parts[:N]
    p1 = parts[ACC_ROWS:ACC_ROWS + N]
    return _tc_final(p0, p1, h, Wr2, Ws2, b2r, Wc1, bc1r, Wc2, bc2r)
